# Initial kernel scaffold; baseline (speedup 1.0000x reference)
import jax
import jax.numpy as jnp
from jax.experimental import pallas as pl


def kernel(x, edge_index, target_node_idx, W, a_src, a_dst, fc_w, fc_b):
    raise NotImplementedError("write your pallas kernel here")


# speedup vs baseline: 351.9669x; 351.9669x over previous
import functools

import jax
import jax.numpy as jnp
from jax import lax
from jax.experimental import pallas as pl
from jax.experimental.pallas import tpu as pltpu
from jax.experimental.pallas import tpu_sc as plsc

N = 100000
E = 1600000
F = 4
H = 4
EMB = 16
HID = 128
B = 1024
L = 16
NC = 2
NS = 16
NW = NC * NS
EPW = E // NW
CH = 2000
NGR = CH // L
NCH = EPW // CH
CAP = 2048
K = 128
ACCW = 80
TRASH = B
ACCR = 1056

_f32 = jnp.float32
_i32 = jnp.int32


def _worker_id():
    return lax.axis_index("s") * NC + lax.axis_index("c")


def _sc_main_body(x_hbm, src_hbm, dst_hbm, tgt_hbm, w_hbm, as_hbm, ad_hbm,
                  acc_out, rep_out,
                  segref, tbuf, adt, csrc, cm, sbuf, dbuf, xrows, msgbuf,
                  idxstage, srcstage, wbuf, asbuf, adbuf, stage, acc_vs):
    cid = lax.axis_index("c")
    sid = lax.axis_index("s")
    wid = _worker_id()
    iota = jnp.arange(L, dtype=_i32)

    pltpu.sync_copy(w_hbm, wbuf)
    pltpu.sync_copy(as_hbm, asbuf)
    pltpu.sync_copy(ad_hbm, adbuf)
    pltpu.sync_copy(tgt_hbm, tbuf)

    def _vsum(v):
        s = v[0]
        for e in range(1, L):
            s = s + v[e]
        return s

    wv = [[wbuf[pl.ds(64 * f + 16 * j, L)] for j in range(H)]
          for f in range(F)]
    a_s = [[_vsum(wv[f][j] * asbuf[pl.ds(16 * j, L)]) for j in range(H)]
           for f in range(F)]
    a_d = [[_vsum(wv[f][j] * adbuf[pl.ds(16 * j, L)]) for j in range(H)]
           for f in range(F)]

    zeros16 = jnp.zeros((L,), _f32)
    for r in range(32):
        for q in range(ACCW // L):
            stage[r, pl.ds(L * q, L)] = zeros16
    pltpu.sync_copy(stage, acc_vs.at[pl.ds(64 * sid, 32)])
    pltpu.sync_copy(stage, acc_vs.at[pl.ds(64 * sid + 32, 32)])

    @pl.when(sid == 0)
    def _():
        pltpu.sync_copy(stage, acc_vs.at[pl.ds(1024, 32)])

    def seg_init(i, _):
        segref[pl.ds(L * i, L)] = jnp.full((L,), -1, _i32)
        return 0
    lax.fori_loop(0, (N // 2) // L, seg_init, 0)

    def seg_set(tt, _):
        t16 = tbuf[pl.ds(L * tt, L)]
        for u in range(L):
            node = t16[u]
            widx = jnp.full((L,), 0, _i32) + (node >> 1)
            word = plsc.load_gather(segref, [widx])[0]
            sh = (node & 1) << 4
            neww = (word & ~(0xFFFF << sh)) | ((L * tt + u) << sh)
            plsc.store_scatter(segref, [widx],
                               jnp.full((L,), 0, _i32) + neww)
        return 0
    lax.fori_loop(0, B // L, seg_set, 0)

    plsc.store_scatter(adt, [iota + 4096], jnp.zeros((L,), _f32))

    def adt_chunk(c8, _):
        pltpu.sync_copy(x_hbm.at[tbuf.at[pl.ds(K * c8, K)]], xrows)

        def group_body(g, _):
            lanes = iota + L * g
            xc = [plsc.load_gather(xrows, [lanes, jnp.full((L,), f, _i32)])
                  for f in range(F)]
            slot = K * c8 + lanes
            for j in range(H):
                v = (xc[0] * a_d[0][j] + xc[1] * a_d[1][j]
                     + xc[2] * a_d[2][j] + xc[3] * a_d[3][j])
                plsc.store_scatter(adt, [4 * slot + j], v)
            return 0
        lax.fori_loop(0, K // L, group_body, 0)
        return 0
    lax.fori_loop(0, B // K, adt_chunk, 0)

    zcols = jnp.zeros((L,), _f32)

    def msg_zero(g, _):
        lanes = iota + L * g
        for cc in range(64, ACCW):
            plsc.store_scatter(msgbuf, [lanes, jnp.full((L,), cc, _i32)],
                               zcols)
        return 0
    lax.fori_loop(0, K // L, msg_zero, 0)

    plsc.subcore_barrier()

    ebase = wid * EPW

    def chunk_body(ci, off):
        pltpu.sync_copy(src_hbm.at[pl.ds(ebase + CH * ci, CH)], sbuf)
        pltpu.sync_copy(dst_hbm.at[pl.ds(ebase + CH * ci, CH)], dbuf)

        def group_body(g, off):
            d16 = dbuf[pl.ds(L * g, L)]
            s16 = sbuf[pl.ds(L * g, L)]
            word = plsc.load_gather(segref, [lax.shift_right_logical(d16,
                                                                     1)])
            sh = (d16 & 1) << 4
            m16 = lax.shift_right_logical(word, sh) & 0xFFFF
            msk = m16 != 0xFFFF
            cnt = plsc.all_reduce_population_count(msk)[0]
            offc = jnp.minimum(off, CAP - L)
            plsc.store_compressed(csrc.at[pl.ds(offc, L)], s16, mask=msk)
            plsc.store_compressed(cm.at[pl.ds(offc, L)], m16, mask=msk)
            return offc + cnt

        return lax.fori_loop(0, NGR, group_body, off)

    nv = lax.fori_loop(0, NCH, chunk_body, jnp.int32(0))

    pbase = (nv // L) * L
    trash16 = jnp.full((L,), TRASH, _i32)
    zero16 = jnp.zeros((L,), _i32)

    def pad_body(k, _):
        idx = iota + pbase + L * k
        idx = jnp.where(idx >= nv, idx, CAP)
        plsc.store_scatter(cm, [idx], trash16)
        plsc.store_scatter(csrc, [idx], zero16)
        return 0
    lax.fori_loop(0, (CAP - pbase) // L, pad_body, 0)

    wsc = [[[wv[f][j][e] for e in range(EMB)] for j in range(H)]
           for f in range(F)]

    def proc_body(c, _):
        def stage_body(g, _):
            srcstage[pl.ds(L * g, L)] = csrc[pl.ds(K * c + L * g, L)]
            idxstage[pl.ds(L * g, L)] = cm[pl.ds(K * c + L * g, L)]
            return 0
        lax.fori_loop(0, K // L, stage_body, 0)
        pltpu.sync_copy(x_hbm.at[srcstage], xrows)

        def group_body(g, _):
            lanes = iota + L * g
            xc = [plsc.load_gather(xrows, [lanes, jnp.full((L,), f, _i32)])
                  for f in range(F)]
            m16 = idxstage[pl.ds(L * g, L)]
            for j in range(H):
                s = (xc[0] * a_s[0][j] + xc[1] * a_s[1][j]
                     + xc[2] * a_s[2][j] + xc[3] * a_s[3][j])
                s = s + plsc.load_gather(adt, [4 * m16 + j])
                e = jnp.maximum(s, 0.2 * s)
                p = jnp.exp(e)
                plsc.store_scatter(msgbuf, [lanes, jnp.full((L,), 64 + j,
                                                            _i32)], p)
                for ec in range(EMB):
                    hcol = (xc[0] * wsc[0][j][ec] + xc[1] * wsc[1][j][ec]
                            + xc[2] * wsc[2][j][ec] + xc[3] * wsc[3][j][ec])
                    plsc.store_scatter(
                        msgbuf,
                        [lanes, jnp.full((L,), EMB * j + ec, _i32)],
                        hcol * p)
            return 0
        lax.fori_loop(0, K // L, group_body, 0)
        for t in range(NS):
            plsc.subcore_barrier()

            @pl.when(sid == t)
            def _():
                pltpu.sync_copy(msgbuf, acc_vs.at[idxstage], add=True)
        return 0

    lax.fori_loop(0, CAP // K, proc_body, 0)

    plsc.subcore_barrier()

    base = 64 * sid

    base2 = ACCR * cid + base
    for half in range(2):
        pltpu.sync_copy(acc_vs.at[pl.ds(base + 32 * half, 32)], stage)
        pltpu.sync_copy(stage, acc_out.at[pl.ds(base2 + 32 * half, 32)])

    @pl.when(jnp.logical_and(cid == 0, sid == 0))
    def _():
        for c8 in range(B // K):
            for g in range(K // L):
                t16 = tbuf[pl.ds(K * c8 + L * g, L)]
                wrd = plsc.load_gather(segref,
                                       [lax.shift_right_logical(t16, 1)])
                shr = (t16 & 1) << 4
                idxstage[pl.ds(L * g, L)] = (lax.shift_right_logical(wrd,
                                                                     shr)
                                             & 0xFFFF)
            pltpu.sync_copy(idxstage, rep_out.at[pl.ds(K * c8, K)])


def _sc_combine_body(acc_hbm, rep_hbm, emb_out,
                     repb, repb2, rows0, rows1, embb):
    wid = _worker_id()
    b0 = 32 * wid
    pltpu.sync_copy(rep_hbm.at[pl.ds(b0, 32)], repb)
    for g in range(2):
        repb2[pl.ds(L * g, L)] = repb[pl.ds(L * g, L)] + ACCR
    pltpu.sync_copy(acc_hbm.at[repb], rows0)
    pltpu.sync_copy(acc_hbm.at[repb2], rows1)

    iota = jnp.arange(L, dtype=_i32)

    def slot_body(g, _):
        lanes = iota + L * g
        for j in range(H):
            cden = jnp.full((L,), 64 + j, _i32)
            den = (plsc.load_gather(rows0, [lanes, cden])
                   + plsc.load_gather(rows1, [lanes, cden]) + 1e-16)
            for ec in range(EMB):
                cc = jnp.full((L,), EMB * j + ec, _i32)
                num = (plsc.load_gather(rows0, [lanes, cc])
                       + plsc.load_gather(rows1, [lanes, cc]))
                plsc.store_scatter(embb, [lanes, cc], num / den)
        return 0
    lax.fori_loop(0, 2, slot_body, 0)
    pltpu.sync_copy(embb, emb_out.at[pl.ds(b0, 32)])


def _SC_MAIN_SCRATCH():
    return [
        pltpu.VMEM((N // 2,), _i32),
        pltpu.VMEM((B,), _i32),
        pltpu.VMEM((4112,), _f32),
        pltpu.VMEM((CAP + L,), _i32),
        pltpu.VMEM((CAP + L,), _i32),
        pltpu.VMEM((CH,), _i32),
        pltpu.VMEM((CH,), _i32),
        pltpu.VMEM((K, L), _f32),
        pltpu.VMEM((K, ACCW), _f32),
        pltpu.VMEM((K,), _i32),
        pltpu.VMEM((K,), _i32),
        pltpu.VMEM((F * H * EMB,), _f32),
        pltpu.VMEM((H * EMB,), _f32),
        pltpu.VMEM((H * EMB,), _f32),
        pltpu.VMEM((32, ACCW), _f32),
        pltpu.VMEM_SHARED((ACCR, ACCW), _f32),
    ]


def _fc_body(emb_ref, w_ref, b_ref, o_ref):
    o_ref[...] = lax.dot_general(
        emb_ref[...], w_ref[...], (((1,), (1,)), ((), ())),
        preferred_element_type=_f32) + b_ref[...]


def kernel(x, edge_index, target_node_idx, W, a_src, a_dst, fc_w, fc_b):
    src = edge_index[0]
    dst = edge_index[1]
    x = jnp.pad(x, ((0, 0), (0, L - F)))
    mesh = plsc.VectorSubcoreMesh(core_axis_name="c", subcore_axis_name="s",
                                  num_cores=NC, num_subcores=NS)

    sc_main = pl.kernel(
        _sc_main_body,
        out_type=[
            jax.ShapeDtypeStruct((2 * ACCR, ACCW), _f32),
            jax.ShapeDtypeStruct((B,), _i32),
        ],
        mesh=mesh,
        compiler_params=pltpu.CompilerParams(needs_layout_passes=False,
                                             use_tc_tiling_on_sc=False),
        scratch_types=_SC_MAIN_SCRATCH(),
    )
    acc, rep = sc_main(x, src, dst, target_node_idx,
                       W.reshape(-1), a_src.reshape(-1),
                       a_dst.reshape(-1))

    sc_comb = pl.kernel(
        _sc_combine_body,
        out_type=jax.ShapeDtypeStruct((B, H * EMB), _f32),
        mesh=mesh,
        compiler_params=pltpu.CompilerParams(needs_layout_passes=False,
                                             use_tc_tiling_on_sc=False),
        scratch_types=[
            pltpu.VMEM((32,), _i32),
            pltpu.VMEM((32,), _i32),
            pltpu.VMEM((32, ACCW), _f32),
            pltpu.VMEM((32, ACCW), _f32),
            pltpu.VMEM((32, H * EMB), _f32),
        ],
    )
    emb = sc_comb(acc, rep)

    y = pl.pallas_call(
        _fc_body,
        out_shape=jax.ShapeDtypeStruct((B, HID), _f32),
    )(emb, fc_w, fc_b.reshape(1, HID))
    return y
